# Initial kernel scaffold; baseline (speedup 1.0000x reference)
import jax
import jax.numpy as jnp
from jax.experimental import pallas as pl


def kernel(x, We1, be1, We2, be2, Wg, Wn, Wt1, bt1, Wt2, bt2):
    raise NotImplementedError("write your pallas kernel here")


# speedup vs baseline: 1.2595x; 1.2595x over previous
import jax
import jax.numpy as jnp
from jax import lax
from jax.experimental import pallas as pl
from jax.experimental.pallas import tpu as pltpu

E = 10
TOPK = 3
B = 1024
D_IN = 1024
D_EXP = 512
T = 2
NEG = -1e30


def _mmoe_kernel(x_ref, We1_ref, be1_ref, We2_ref, be2_ref,
                 Wg_ref, Wn_ref, noise_ref, Wt1_ref, bt1_ref, Wt2_ref, bt2_ref,
                 out0_ref, out1_ref, acc_ref, g_ref):
    e = pl.program_id(0)
    x = x_ref[...]

    @pl.when(e == 0)
    def _gates():
        gsum = jnp.zeros((B, E), jnp.float32)
        iota = lax.broadcasted_iota(jnp.int32, (B, E), 1)
        for i in range(T):
            mean = jnp.dot(x, Wg_ref[i], preferred_element_type=jnp.float32)
            std = jax.nn.softplus(
                jnp.dot(x, Wn_ref[i], preferred_element_type=jnp.float32))
            H = mean + noise_ref[i] * std
            Hw = H
            for _ in range(TOPK - 1):
                m = jnp.max(Hw, axis=1, keepdims=True)
                idx = jnp.min(jnp.where(Hw == m, iota, E), axis=1,
                              keepdims=True)
                Hw = jnp.where(iota == idx, NEG, Hw)
            thresh = jnp.max(Hw, axis=1, keepdims=True)
            Hm = jnp.where(H < thresh, NEG, H)
            mx = jnp.max(Hm, axis=1, keepdims=True)
            p = jnp.exp(Hm - mx)
            gsum = gsum + p / jnp.sum(p, axis=1, keepdims=True)
        g_ref[...] = gsum

    h = jnp.maximum(
        jnp.dot(x, We1_ref[0], preferred_element_type=jnp.float32)
        + be1_ref[0], 0.0)
    f = (jnp.dot(h, We2_ref[0], preferred_element_type=jnp.float32)
         + be2_ref[0])
    onehot = (lax.broadcasted_iota(jnp.int32, (E, 1), 0) == e).astype(
        jnp.float32)
    gcol = jnp.dot(g_ref[...], onehot, preferred_element_type=jnp.float32)
    contrib = gcol * f

    @pl.when(e == 0)
    def _init():
        acc_ref[...] = contrib

    @pl.when(e > 0)
    def _accum():
        acc_ref[...] += contrib

    @pl.when(e == E - 1)
    def _heads():
        shared = acc_ref[...]
        h0 = jnp.maximum(
            jnp.dot(shared, Wt1_ref[0], preferred_element_type=jnp.float32)
            + bt1_ref[0], 0.0)
        out0_ref[...] = (
            jnp.dot(h0, Wt2_ref[0], preferred_element_type=jnp.float32)
            + bt2_ref[0])
        h1 = jnp.maximum(
            jnp.dot(shared, Wt1_ref[1], preferred_element_type=jnp.float32)
            + bt1_ref[1], 0.0)
        out1_ref[...] = (
            jnp.dot(h1, Wt2_ref[1], preferred_element_type=jnp.float32)
            + bt2_ref[1])


def _const0(*_):
    return 0


@jax.jit
def kernel(x, We1, be1, We2, be2, Wg, Wn, Wt1, bt1, Wt2, bt2):
    nkey = jax.random.key(42)
    noise = jnp.stack([
        jax.random.normal(jax.random.fold_in(nkey, i), (B, E),
                          dtype=jnp.float32)
        for i in range(T)])

    be1 = be1.reshape(E, 1, D_EXP)
    be2 = be2.reshape(E, 1, D_EXP)
    full = lambda s: pl.BlockSpec(s, lambda e: tuple(0 for _ in s))
    in_specs = [
        full((B, D_IN)),
        pl.BlockSpec((1, D_IN, D_EXP), lambda e: (e, 0, 0)),
        pl.BlockSpec((1, 1, D_EXP), lambda e: (e, 0, 0)),
        pl.BlockSpec((1, D_EXP, D_EXP), lambda e: (e, 0, 0)),
        pl.BlockSpec((1, 1, D_EXP), lambda e: (e, 0, 0)),
        full((T, D_IN, E)),
        full((T, D_IN, E)),
        full((T, B, E)),
        full((T, D_EXP, 512)),
        full((T, 512)),
        full((T, 512, 256)),
        full((T, 256)),
    ]
    out_specs = (full((B, 256)), full((B, 256)))
    out0, out1 = pl.pallas_call(
        _mmoe_kernel,
        grid=(E,),
        in_specs=in_specs,
        out_specs=out_specs,
        out_shape=(jax.ShapeDtypeStruct((B, 256), jnp.float32),
                   jax.ShapeDtypeStruct((B, 256), jnp.float32)),
        scratch_shapes=[pltpu.VMEM((B, D_EXP), jnp.float32),
                        pltpu.VMEM((B, E), jnp.float32)],
        compiler_params=pltpu.CompilerParams(
            dimension_semantics=("arbitrary",)),
    )(x, We1, be1, We2, be2, Wg, Wn, noise, Wt1, bt1, Wt2, bt2)
    return (out0, out1)


# speedup vs baseline: 1.2700x; 1.0083x over previous
import jax
import jax.numpy as jnp
from jax import lax
from jax.experimental import pallas as pl
from jax.experimental.pallas import tpu as pltpu

E = 10
TOPK = 3
B = 1024
D_IN = 1024
D_EXP = 512
T = 2
NEG = -1e30


def _mmoe_kernel(x_ref, We1_ref, be1_ref, We2_ref, be2_ref,
                 Wg_ref, Wn_ref, noise_ref, Wt1_ref, bt1_ref, Wt2_ref, bt2_ref,
                 out0_ref, out1_ref, acc_ref, g_ref):
    e = pl.program_id(0)
    x = x_ref[...]

    @pl.when(e == 0)
    def _gates():
        gsum = jnp.zeros((B, E), jnp.float32)
        iota = lax.broadcasted_iota(jnp.int32, (B, E), 1)
        for i in range(T):
            mean = jnp.dot(x, Wg_ref[i], preferred_element_type=jnp.float32)
            std = jax.nn.softplus(
                jnp.dot(x, Wn_ref[i], preferred_element_type=jnp.float32))
            H = mean + noise_ref[i] * std
            Hw = H
            for _ in range(TOPK - 1):
                m = jnp.max(Hw, axis=1, keepdims=True)
                idx = jnp.min(jnp.where(Hw == m, iota, E), axis=1,
                              keepdims=True)
                Hw = jnp.where(iota == idx, NEG, Hw)
            thresh = jnp.max(Hw, axis=1, keepdims=True)
            Hm = jnp.where(H < thresh, NEG, H)
            mx = jnp.max(Hm, axis=1, keepdims=True)
            p = jnp.exp(Hm - mx)
            gsum = gsum + p / jnp.sum(p, axis=1, keepdims=True)
        g_ref[...] = gsum

    h = jnp.maximum(
        jnp.dot(x.astype(jnp.bfloat16), We1_ref[0].astype(jnp.bfloat16),
                preferred_element_type=jnp.float32)
        + be1_ref[0], 0.0)
    f = (jnp.dot(h.astype(jnp.bfloat16), We2_ref[0].astype(jnp.bfloat16),
                 preferred_element_type=jnp.float32)
         + be2_ref[0])
    onehot = (lax.broadcasted_iota(jnp.int32, (E, 1), 0) == e).astype(
        jnp.float32)
    gcol = jnp.dot(g_ref[...], onehot, preferred_element_type=jnp.float32)
    contrib = gcol * f

    @pl.when(e == 0)
    def _init():
        acc_ref[...] = contrib

    @pl.when(e > 0)
    def _accum():
        acc_ref[...] += contrib

    @pl.when(e == E - 1)
    def _heads():
        shared = acc_ref[...].astype(jnp.bfloat16)
        h0 = jnp.maximum(
            jnp.dot(shared, Wt1_ref[0].astype(jnp.bfloat16),
                    preferred_element_type=jnp.float32)
            + bt1_ref[0], 0.0)
        out0_ref[...] = (
            jnp.dot(h0.astype(jnp.bfloat16), Wt2_ref[0].astype(jnp.bfloat16),
                    preferred_element_type=jnp.float32)
            + bt2_ref[0])
        h1 = jnp.maximum(
            jnp.dot(shared, Wt1_ref[1].astype(jnp.bfloat16),
                    preferred_element_type=jnp.float32)
            + bt1_ref[1], 0.0)
        out1_ref[...] = (
            jnp.dot(h1.astype(jnp.bfloat16), Wt2_ref[1].astype(jnp.bfloat16),
                    preferred_element_type=jnp.float32)
            + bt2_ref[1])


def _const0(*_):
    return 0


@jax.jit
def kernel(x, We1, be1, We2, be2, Wg, Wn, Wt1, bt1, Wt2, bt2):
    nkey = jax.random.key(42)
    noise = jnp.stack([
        jax.random.normal(jax.random.fold_in(nkey, i), (B, E),
                          dtype=jnp.float32)
        for i in range(T)])

    be1 = be1.reshape(E, 1, D_EXP)
    be2 = be2.reshape(E, 1, D_EXP)
    full = lambda s: pl.BlockSpec(s, lambda e: tuple(0 for _ in s))
    in_specs = [
        full((B, D_IN)),
        pl.BlockSpec((1, D_IN, D_EXP), lambda e: (e, 0, 0)),
        pl.BlockSpec((1, 1, D_EXP), lambda e: (e, 0, 0)),
        pl.BlockSpec((1, D_EXP, D_EXP), lambda e: (e, 0, 0)),
        pl.BlockSpec((1, 1, D_EXP), lambda e: (e, 0, 0)),
        full((T, D_IN, E)),
        full((T, D_IN, E)),
        full((T, B, E)),
        full((T, D_EXP, 512)),
        full((T, 512)),
        full((T, 512, 256)),
        full((T, 256)),
    ]
    out_specs = (full((B, 256)), full((B, 256)))
    out0, out1 = pl.pallas_call(
        _mmoe_kernel,
        grid=(E,),
        in_specs=in_specs,
        out_specs=out_specs,
        out_shape=(jax.ShapeDtypeStruct((B, 256), jnp.float32),
                   jax.ShapeDtypeStruct((B, 256), jnp.float32)),
        scratch_shapes=[pltpu.VMEM((B, D_EXP), jnp.float32),
                        pltpu.VMEM((B, E), jnp.float32)],
        compiler_params=pltpu.CompilerParams(
            dimension_semantics=("arbitrary",)),
    )(x, We1, be1, We2, be2, Wg, Wn, noise, Wt1, bt1, Wt2, bt2)
    return (out0, out1)


# speedup vs baseline: 1.6047x; 1.2635x over previous
import jax
import jax.numpy as jnp
from jax import lax
from jax.experimental import pallas as pl
from jax.experimental.pallas import tpu as pltpu

E = 10
TOPK = 3
B = 1024
D_IN = 1024
D_EXP = 512
T = 2
NEG = -1e30
HALF = B // 2


def _mmoe_kernel(x_ref, We1_ref, be1_ref, We2_ref, be2_ref,
                 Wg_ref, Wn_ref, noise_ref, Wt1_ref, bt1_ref, Wt2_ref, bt2_ref,
                 out0_ref, out1_ref, acc_ref, g_ref, xb_ref):
    e = pl.program_id(0)

    @pl.when(e == 0)
    def _prologue():
        x = x_ref[...]
        xb_ref[...] = x.astype(jnp.bfloat16)
        gsum = jnp.zeros((B, E), jnp.float32)
        iota = lax.broadcasted_iota(jnp.int32, (B, E), 1)
        for i in range(T):
            mean = jnp.dot(x, Wg_ref[i], preferred_element_type=jnp.float32)
            std = jax.nn.softplus(
                jnp.dot(x, Wn_ref[i], preferred_element_type=jnp.float32))
            H = mean + noise_ref[i] * std
            Hw = H
            for _ in range(TOPK - 1):
                m = jnp.max(Hw, axis=1, keepdims=True)
                idx = jnp.min(jnp.where(Hw == m, iota, E), axis=1,
                              keepdims=True)
                Hw = jnp.where(iota == idx, NEG, Hw)
            thresh = jnp.max(Hw, axis=1, keepdims=True)
            Hm = jnp.where(H < thresh, NEG, H)
            mx = jnp.max(Hm, axis=1, keepdims=True)
            p = jnp.exp(Hm - mx)
            gsum = gsum + p / jnp.sum(p, axis=1, keepdims=True)
        g_ref[...] = gsum

    onehot = (lax.broadcasted_iota(jnp.int32, (E, 1), 0) == e).astype(
        jnp.float32)
    gcol = jnp.dot(g_ref[...], onehot, preferred_element_type=jnp.float32)
    We1 = We1_ref[0]
    We2 = We2_ref[0]
    be1 = be1_ref[0]
    be2 = be2_ref[0]
    for s in range(2):
        rows = slice(s * HALF, (s + 1) * HALF)
        h = jnp.maximum(
            jnp.dot(xb_ref[rows, :], We1.astype(jnp.bfloat16),
                    preferred_element_type=jnp.float32) + be1, 0.0)
        f = (jnp.dot(h.astype(jnp.bfloat16), We2.astype(jnp.bfloat16),
                     preferred_element_type=jnp.float32) + be2)
        contrib = gcol[rows, :] * f

        @pl.when(e == 0)
        def _init():
            acc_ref[rows, :] = contrib

        @pl.when(e > 0)
        def _accum():
            acc_ref[rows, :] += contrib

    @pl.when(e == E - 1)
    def _heads():
        shared = acc_ref[...].astype(jnp.bfloat16)
        for t, out_ref in ((0, out0_ref), (1, out1_ref)):
            ht = jnp.maximum(
                jnp.dot(shared, Wt1_ref[t].astype(jnp.bfloat16),
                        preferred_element_type=jnp.float32)
                + bt1_ref[t], 0.0)
            out_ref[...] = (
                jnp.dot(ht.astype(jnp.bfloat16),
                        Wt2_ref[t].astype(jnp.bfloat16),
                        preferred_element_type=jnp.float32)
                + bt2_ref[t])


@jax.jit
def kernel(x, We1, be1, We2, be2, Wg, Wn, Wt1, bt1, Wt2, bt2):
    with jax.ensure_compile_time_eval():
        nkey = jax.random.key(42)
        noise = jnp.stack([
            jax.random.normal(jax.random.fold_in(nkey, i), (B, E),
                              dtype=jnp.float32)
            for i in range(T)])

    be1 = be1.reshape(E, 1, D_EXP)
    be2 = be2.reshape(E, 1, D_EXP)
    full = lambda s: pl.BlockSpec(s, lambda e: tuple(0 for _ in s))
    in_specs = [
        full((B, D_IN)),
        pl.BlockSpec((1, D_IN, D_EXP), lambda e: (e, 0, 0)),
        pl.BlockSpec((1, 1, D_EXP), lambda e: (e, 0, 0)),
        pl.BlockSpec((1, D_EXP, D_EXP), lambda e: (e, 0, 0)),
        pl.BlockSpec((1, 1, D_EXP), lambda e: (e, 0, 0)),
        full((T, D_IN, E)),
        full((T, D_IN, E)),
        full((T, B, E)),
        full((T, D_EXP, 512)),
        full((T, 512)),
        full((T, 512, 256)),
        full((T, 256)),
    ]
    out_specs = (full((B, 256)), full((B, 256)))
    out0, out1 = pl.pallas_call(
        _mmoe_kernel,
        grid=(E,),
        in_specs=in_specs,
        out_specs=out_specs,
        out_shape=(jax.ShapeDtypeStruct((B, 256), jnp.float32),
                   jax.ShapeDtypeStruct((B, 256), jnp.float32)),
        scratch_shapes=[pltpu.VMEM((B, D_EXP), jnp.float32),
                        pltpu.VMEM((B, E), jnp.float32),
                        pltpu.VMEM((B, D_IN), jnp.bfloat16)],
        compiler_params=pltpu.CompilerParams(
            dimension_semantics=("arbitrary",)),
    )(x, We1, be1, We2, be2, Wg, Wn, noise, Wt1, bt1, Wt2, bt2)
    return (out0, out1)
